# Initial kernel scaffold; baseline (speedup 1.0000x reference)
import jax
import jax.numpy as jnp
from jax.experimental import pallas as pl


def kernel(x, table):
    raise NotImplementedError("write your pallas kernel here")


# speedup vs baseline: 2.6220x; 2.6220x over previous
import functools

import jax
import jax.numpy as jnp
from jax import lax
from jax.experimental import pallas as pl
from jax.experimental.pallas import tpu as pltpu
from jax.experimental.pallas import tpu_sc as plsc


_NC, _NS, _NL = 2, 16, 16


@functools.partial(jax.jit, static_argnames=("b", "l", "d"))
def _shifted_embed(x, table, b, l, d):
  nc, ns, nl = _NC, _NS, _NL
  nw = nc * ns
  assert b % nw == 0
  rows_per_w = b // nw

  mesh = plsc.VectorSubcoreMesh(
      core_axis_name="c", subcore_axis_name="s", num_cores=nc, num_subcores=ns)

  @functools.partial(
      pl.kernel,
      out_type=jax.ShapeDtypeStruct((b * l, d), jnp.float32),
      mesh=mesh,
      scratch_types=[
          pltpu.VMEM((rows_per_w, l), jnp.int32),
          pltpu.VMEM((l + 1, d), jnp.float32),
          pltpu.SemaphoreType.DMA,
      ],
      compiler_params=pltpu.CompilerParams(use_tc_tiling_on_sc=False),
  )
  def k(x_hbm, table_hbm, out_hbm, idx_v, rows_v, sem):
    wid = lax.axis_index("s") * nc + lax.axis_index("c")
    b0 = wid * rows_per_w
    pltpu.sync_copy(x_hbm.at[pl.ds(b0, rows_per_w)], idx_v)
    zeros = jnp.zeros((nl,), jnp.float32)
    for c in range(d // nl):
      rows_v[l, pl.ds(c * nl, nl)] = zeros

    def body(r, _):
      pltpu.async_copy(table_hbm.at[idx_v.at[r]], rows_v.at[pl.ds(0, l)],
                       sem).wait()
      pltpu.sync_copy(rows_v.at[pl.ds(1, l)],
                      out_hbm.at[pl.ds((b0 + r) * l, l)])
      return _

    lax.fori_loop(0, rows_per_w, body, None)

  return k(x, table)


def kernel(x, table):
  b, l = x.shape
  v, d = table.shape
  out = _shifted_embed(x.astype(jnp.int32), table, b, l, d)
  return out.reshape(b, l, d)


# speedup vs baseline: 3.4730x; 1.3246x over previous
import functools

import jax
import jax.numpy as jnp
from jax import lax
from jax.experimental import pallas as pl
from jax.experimental.pallas import tpu as pltpu
from jax.experimental.pallas import tpu_sc as plsc

_NC, _NS, _NL = 2, 16, 16
_NBUF = 4
_ROWS_PER_CHUNK = 2


@functools.partial(jax.jit, static_argnames=("b", "l", "d"))
def _shifted_embed(x2, table, b, l, d):
  nc, ns, nl = _NC, _NS, _NL
  nw = nc * ns
  chunk = _ROWS_PER_CHUNK * l
  assert chunk <= 128
  n_chunks = (b * l) // chunk
  assert n_chunks % (nw * _NBUF) == 0
  cpw = n_chunks // nw
  n_outer = cpw // _NBUF

  mesh = plsc.VectorSubcoreMesh(
      core_axis_name="c", subcore_axis_name="s", num_cores=nc, num_subcores=ns)

  buf_t = pltpu.VMEM((chunk + 1, d), jnp.float32)

  @functools.partial(
      pl.kernel,
      out_type=jax.ShapeDtypeStruct((b * l, d), jnp.float32),
      mesh=mesh,
      scratch_types=[
          pltpu.VMEM((cpw, chunk), jnp.int32),
          [buf_t] * _NBUF,
          [pltpu.SemaphoreType.DMA] * _NBUF,
          [pltpu.SemaphoreType.DMA] * _NBUF,
      ],
      compiler_params=pltpu.CompilerParams(use_tc_tiling_on_sc=False),
  )
  def k(x_hbm, table_hbm, out_hbm, idx_v, bufs, gsems, osems):
    wid = lax.axis_index("s") * nc + lax.axis_index("c")
    s0 = wid * cpw * chunk
    pltpu.sync_copy(x_hbm.at[pl.ds(wid * cpw, cpw)], idx_v)
    zeros = jnp.zeros((nl,), jnp.float32)
    for q in range(_NBUF):
      for c in range(d // nl):
        bufs[q][chunk, pl.ds(c * nl, nl)] = zeros
      pltpu.async_copy(table_hbm.at[idx_v.at[q]],
                       bufs[q].at[pl.ds(0, chunk)], gsems[q])

    def body(t, _):
      for q in range(_NBUF):
        j = t * _NBUF + q
        buf, gsem, osem = bufs[q], gsems[q], osems[q]
        pltpu.make_async_copy(table_hbm.at[idx_v.at[j]],
                              buf.at[pl.ds(0, chunk)], gsem).wait()
        for c in range(d // nl):
          buf[l, pl.ds(c * nl, nl)] = zeros
        pltpu.async_copy(buf.at[pl.ds(1, chunk)],
                         out_hbm.at[pl.ds(s0 + j * chunk, chunk)], osem)
      for q in range(_NBUF):
        j = t * _NBUF + q
        buf, gsem, osem = bufs[q], gsems[q], osems[q]
        pltpu.make_async_copy(buf.at[pl.ds(1, chunk)],
                              out_hbm.at[pl.ds(s0 + j * chunk, chunk)],
                              osem).wait()

        @pl.when(t < n_outer - 1)
        def _():
          pltpu.async_copy(table_hbm.at[idx_v.at[j + _NBUF]],
                           buf.at[pl.ds(0, chunk)], gsem)

      return _

    lax.fori_loop(0, n_outer, body, None)

  return k(x2, table)


def kernel(x, table):
  b, l = x.shape
  v, d = table.shape
  chunk = _ROWS_PER_CHUNK * l
  x2 = x.astype(jnp.int32).reshape((b * l) // chunk, chunk)
  out = _shifted_embed(x2, table, b, l, d)
  return out.reshape(b, l, d)


# speedup vs baseline: 5.7690x; 1.6611x over previous
import functools

import jax
import jax.numpy as jnp
from jax import lax
from jax.experimental import pallas as pl
from jax.experimental.pallas import tpu as pltpu
from jax.experimental.pallas import tpu_sc as plsc

_NC, _NS, _NL = 2, 16, 16
_NBUF = 4
_NR = 2


@functools.partial(jax.jit, static_argnames=("b", "l", "d"))
def _shifted_embed(x, table, b, l, d):
  nc, ns, nl = _NC, _NS, _NL
  nw = nc * ns
  rpw = b // nw
  assert rpw % (_NR * _NBUF) == 0
  n_chunks = rpw // _NR
  n_outer = n_chunks // _NBUF
  lpad = ((l + 7) // 8) * 8
  sidx_len = rpw * lpad + 2 * nl

  mesh = plsc.VectorSubcoreMesh(
      core_axis_name="c", subcore_axis_name="s", num_cores=nc, num_subcores=ns)

  buf_t = pltpu.VMEM((_NR, l, d), jnp.float32)

  @functools.partial(
      pl.kernel,
      out_type=jax.ShapeDtypeStruct((b, l, d), jnp.float32),
      mesh=mesh,
      scratch_types=[
          pltpu.VMEM((rpw, l), jnp.int32),
          pltpu.VMEM((sidx_len,), jnp.int32),
          [buf_t] * _NBUF,
          [pltpu.SemaphoreType.DMA] * _NBUF,
          [pltpu.SemaphoreType.DMA] * _NBUF,
      ],
      compiler_params=pltpu.CompilerParams(needs_layout_passes=False),
  )
  def k(x_hbm, table_hbm, z_hbm, out_hbm, xv, sidx, bufs, gsems, osems):
    wid = lax.axis_index("s") * nc + lax.axis_index("c")
    b0 = wid * rpw
    pltpu.sync_copy(x_hbm.at[pl.ds(b0, rpw)], xv)

    for q in range(_NBUF):
      for bb in range(_NR):
        pltpu.sync_copy(z_hbm, bufs[q].at[bb].at[pl.ds(l - 1, 1)])

    lane = lax.iota(jnp.int32, nl)
    nsub = (l + nl - 1) // nl

    def build(r, _):
      for c in range(nsub):
        col = jnp.minimum(lane + (c * nl + 1), l - 1)
        vals = plsc.load_gather(xv, [jnp.full((nl,), r, jnp.int32), col])
        sidx[pl.ds(r * lpad + c * nl, nl)] = vals
      return _

    lax.fori_loop(0, rpw, build, None)

    def fire(chunk_i, q):
      for bb in range(_NR):
        r = chunk_i * _NR + bb
        pltpu.async_copy(table_hbm.at[sidx.at[pl.ds(r * lpad, l - 1)]],
                         bufs[q].at[bb].at[pl.ds(0, l - 1)], gsems[q])

    for q in range(_NBUF):
      fire(q, q)

    def body(t, _):
      for q in range(_NBUF):
        j = t * _NBUF + q
        buf, gsem, osem = bufs[q], gsems[q], osems[q]
        for bb in range(_NR):
          pltpu.make_async_copy(table_hbm.at[sidx.at[pl.ds(0, l - 1)]],
                                buf.at[bb].at[pl.ds(0, l - 1)], gsem).wait()
        pltpu.async_copy(buf, out_hbm.at[pl.ds(b0 + j * _NR, _NR)], osem)
      for q in range(_NBUF):
        j = t * _NBUF + q
        buf, osem = bufs[q], osems[q]
        pltpu.make_async_copy(buf, out_hbm.at[pl.ds(b0 + j * _NR, _NR)],
                              osem).wait()

        @pl.when(t < n_outer - 1)
        def _():
          fire(j + _NBUF, q)

      return _

    lax.fori_loop(0, n_outer, body, None)

  return k(x, table, jnp.zeros((1, d), jnp.float32))


def kernel(x, table):
  b, l = x.shape
  v, d = table.shape
  out = _shifted_embed(x.astype(jnp.int32), table, b, l, d)
  return out


# speedup vs baseline: 10.4658x; 1.8142x over previous
import functools

import jax
import jax.numpy as jnp
from jax import lax
from jax.experimental import pallas as pl
from jax.experimental.pallas import tpu as pltpu
from jax.experimental.pallas import tpu_sc as plsc

_NC, _NS, _NL = 2, 16, 16
_NBUF = 4


@functools.partial(jax.jit, static_argnames=("b", "l", "d"))
def _shifted_embed(xt, table, b, l, d):
  nc, ns, nl = _NC, _NS, _NL
  nw = nc * ns
  bw = b // nw
  p = l - 1
  assert bw % 128 == 0 and d % nl == 0

  mesh = plsc.VectorSubcoreMesh(
      core_axis_name="c", subcore_axis_name="s", num_cores=nc, num_subcores=ns)

  buf_t = pltpu.VMEM((bw, d), jnp.float32)

  @functools.partial(
      pl.kernel,
      out_type=jax.ShapeDtypeStruct((l, b, d), jnp.float32),
      mesh=mesh,
      scratch_types=[
          pltpu.VMEM((l, bw), jnp.int32),
          pltpu.VMEM((p * bw,), jnp.int32),
          [buf_t] * _NBUF,
          [pltpu.SemaphoreType.DMA] * _NBUF,
          [pltpu.SemaphoreType.DMA] * _NBUF,
      ],
      compiler_params=pltpu.CompilerParams(needs_layout_passes=False),
  )
  def k(xt_hbm, table_hbm, z_hbm, out_hbm, xv, cidx, bufs, gsems, osems):
    wid = lax.axis_index("s") * nc + lax.axis_index("c")
    b0 = wid * bw
    pltpu.sync_copy(xt_hbm.at[:, pl.ds(b0, bw)], xv)

    pltpu.sync_copy(z_hbm, bufs[0])
    pltpu.sync_copy(bufs[0], out_hbm.at[l - 1].at[pl.ds(b0, bw)])

    lane = lax.iota(jnp.int32, nl)

    def build(li, _):
      for c in range(bw // nl):
        vals = plsc.load_gather(
            xv, [jnp.full((nl,), li + 1, jnp.int32), lane + c * nl])
        cidx[pl.ds(li * bw + c * nl, nl)] = vals
      return _

    lax.fori_loop(0, p, build, None)

    def fire(li, q):
      pltpu.async_copy(table_hbm.at[cidx.at[pl.ds(li * bw, bw)]],
                       bufs[q], gsems[q])

    for q in range(_NBUF):
      fire(q, q)

    n_outer = p // _NBUF

    def body(t, _):
      for q in range(_NBUF):
        j = t * _NBUF + q
        pltpu.make_async_copy(table_hbm.at[cidx.at[pl.ds(0, bw)]],
                              bufs[q], gsems[q]).wait()
        pltpu.async_copy(bufs[q], _plane_dst(out_hbm, j, b0), osems[q])
      for q in range(_NBUF):
        j = t * _NBUF + q
        pltpu.make_async_copy(bufs[q], _plane_dst(out_hbm, j, b0),
                              osems[q]).wait()

        @pl.when(j + _NBUF < p)
        def _():
          fire(j + _NBUF, q)

      return _

    def _plane_dst(out_ref, li, base):
      return out_ref.at[li].at[pl.ds(base, bw)]

    lax.fori_loop(0, n_outer, body, None)

    last_q = 0
    pltpu.make_async_copy(table_hbm.at[cidx.at[pl.ds(0, bw)]],
                          bufs[last_q], gsems[last_q]).wait()
    pltpu.sync_copy(bufs[last_q], _plane_dst(out_hbm, p - 1, b0))

  return k(xt, table, jnp.zeros((b // nw, d), jnp.float32))


def kernel(x, table):
  b, l = x.shape
  v, d = table.shape
  out = _shifted_embed(x.astype(jnp.int32).T, table, b, l, d)
  return out.transpose(1, 0, 2)


# speedup vs baseline: 10.5788x; 1.0108x over previous
import functools

import jax
import jax.numpy as jnp
from jax import lax
from jax.experimental import pallas as pl
from jax.experimental.pallas import tpu as pltpu
from jax.experimental.pallas import tpu_sc as plsc

_NC, _NS, _NL = 2, 16, 16
_NBUF = 6


@functools.partial(jax.jit, static_argnames=("b", "l", "d"))
def _shifted_embed(xt, table, b, l, d):
  nc, ns, nl = _NC, _NS, _NL
  nw = nc * ns
  bw = b // nw
  p = l - 1
  assert bw % 128 == 0 and d % nl == 0

  mesh = plsc.VectorSubcoreMesh(
      core_axis_name="c", subcore_axis_name="s", num_cores=nc, num_subcores=ns)

  buf_t = pltpu.VMEM((bw, d), jnp.float32)

  @functools.partial(
      pl.kernel,
      out_type=jax.ShapeDtypeStruct((l, b, d), jnp.float32),
      mesh=mesh,
      scratch_types=[
          pltpu.VMEM((l, bw), jnp.int32),
          pltpu.VMEM((p * bw,), jnp.int32),
          [buf_t] * _NBUF,
          [pltpu.SemaphoreType.DMA] * _NBUF,
          [pltpu.SemaphoreType.DMA] * _NBUF,
      ],
      compiler_params=pltpu.CompilerParams(needs_layout_passes=False),
  )
  def k(xt_hbm, table_hbm, z_hbm, out_hbm, xv, cidx, bufs, gsems, osems):
    wid = lax.axis_index("s") * nc + lax.axis_index("c")
    b0 = wid * bw
    pltpu.sync_copy(xt_hbm.at[:, pl.ds(b0, bw)], xv)

    pltpu.sync_copy(z_hbm, bufs[0])
    pltpu.sync_copy(bufs[0], out_hbm.at[l - 1].at[pl.ds(b0, bw)])

    lane = lax.iota(jnp.int32, nl)

    def build(li):
      for c in range(bw // nl):
        vals = plsc.load_gather(
            xv, [jnp.full((nl,), li + 1, jnp.int32), lane + c * nl])
        cidx[pl.ds(li * bw + c * nl, nl)] = vals

    def fire(li, q):
      pltpu.async_copy(table_hbm.at[cidx.at[pl.ds(li * bw, bw)]],
                       bufs[q], gsems[q])

    for q in range(_NBUF):
      build(q)
      fire(q, q)

    n_outer = p // _NBUF

    def body(t, _):
      for q in range(_NBUF):
        j = t * _NBUF + q
        pltpu.make_async_copy(table_hbm.at[cidx.at[pl.ds(0, bw)]],
                              bufs[q], gsems[q]).wait()
        pltpu.async_copy(bufs[q], _plane_dst(out_hbm, j, b0), osems[q])
      for q in range(_NBUF):
        j = t * _NBUF + q
        pltpu.make_async_copy(bufs[q], _plane_dst(out_hbm, j, b0),
                              osems[q]).wait()

        @pl.when(j + _NBUF < p)
        def _():
          build(j + _NBUF)
          fire(j + _NBUF, q)

      return _

    def _plane_dst(out_ref, li, base):
      return out_ref.at[li].at[pl.ds(base, bw)]

    lax.fori_loop(0, n_outer, body, None)

    last_q = 0
    pltpu.make_async_copy(table_hbm.at[cidx.at[pl.ds(0, bw)]],
                          bufs[last_q], gsems[last_q]).wait()
    pltpu.sync_copy(bufs[last_q], _plane_dst(out_hbm, p - 1, b0))

  return k(xt, table, jnp.zeros((b // nw, d), jnp.float32))


def kernel(x, table):
  b, l = x.shape
  v, d = table.shape
  out = _shifted_embed(x.astype(jnp.int32).T, table, b, l, d)
  return out.transpose(1, 0, 2)


# speedup vs baseline: 10.6561x; 1.0073x over previous
import functools

import jax
import jax.numpy as jnp
from jax import lax
from jax.experimental import pallas as pl
from jax.experimental.pallas import tpu as pltpu
from jax.experimental.pallas import tpu_sc as plsc

_NC, _NS, _NL = 2, 16, 16
_NBUF = 7


@functools.partial(jax.jit, static_argnames=("b", "l", "d"))
def _shifted_embed(xt, table, b, l, d):
  nc, ns, nl = _NC, _NS, _NL
  nw = nc * ns
  bw = b // nw
  p = l - 1
  assert bw % 128 == 0 and d % nl == 0 and p % _NBUF == 0

  mesh = plsc.VectorSubcoreMesh(
      core_axis_name="c", subcore_axis_name="s", num_cores=nc, num_subcores=ns)

  buf_t = pltpu.VMEM((bw, d), jnp.float32)

  @functools.partial(
      pl.kernel,
      out_type=jax.ShapeDtypeStruct((l, b, d), jnp.float32),
      mesh=mesh,
      scratch_types=[
          pltpu.VMEM((l, bw), jnp.int32),
          pltpu.VMEM((_NBUF * bw,), jnp.int32),
          [buf_t] * _NBUF,
          [pltpu.SemaphoreType.DMA] * _NBUF,
          [pltpu.SemaphoreType.DMA] * _NBUF,
      ],
      compiler_params=pltpu.CompilerParams(needs_layout_passes=False),
  )
  def k(xt_hbm, table_hbm, z_hbm, out_hbm, xv, cidx, bufs, gsems, osems):
    wid = lax.axis_index("s") * nc + lax.axis_index("c")
    b0 = wid * bw

    def _plane_dst(li):
      return out_hbm.at[li].at[pl.ds(b0, bw)]

    pltpu.async_copy(z_hbm, bufs[0], gsems[0])
    pltpu.sync_copy(xt_hbm.at[:, pl.ds(b0, bw)], xv)
    pltpu.make_async_copy(z_hbm, bufs[0], gsems[0]).wait()
    pltpu.async_copy(bufs[0], _plane_dst(l - 1), osems[0])

    lane = lax.iota(jnp.int32, nl)

    def build(li, q):
      for c in range(bw // nl):
        vals = plsc.load_gather(
            xv, [jnp.full((nl,), li + 1, jnp.int32), lane + c * nl])
        cidx[pl.ds(q * bw + c * nl, nl)] = vals

    def fire(li, q):
      pltpu.async_copy(table_hbm.at[cidx.at[pl.ds(q * bw, bw)]],
                       bufs[q], gsems[q])

    for q in range(_NBUF):
      build(q, q)
      if q == 0:
        pltpu.make_async_copy(bufs[0], _plane_dst(l - 1), osems[0]).wait()
      fire(q, q)

    def body(t, _):
      for q in range(_NBUF):
        j = t * _NBUF + q
        pltpu.make_async_copy(table_hbm.at[cidx.at[pl.ds(0, bw)]],
                              bufs[q], gsems[q]).wait()
        pltpu.async_copy(bufs[q], _plane_dst(j), osems[q])
      for q in range(_NBUF):
        j = t * _NBUF + q
        pltpu.make_async_copy(bufs[q], _plane_dst(j), osems[q]).wait()

        @pl.when(j + _NBUF < p)
        def _():
          build(j + _NBUF, q)
          fire(j + _NBUF, q)

      return _

    lax.fori_loop(0, p // _NBUF, body, None)

  return k(xt, table, jnp.zeros((b // nw, d), jnp.float32))


def kernel(x, table):
  b, l = x.shape
  v, d = table.shape
  out = _shifted_embed(x.astype(jnp.int32).T, table, b, l, d)
  return out.transpose(1, 0, 2)


# speedup vs baseline: 10.6624x; 1.0006x over previous
import functools

import jax
import jax.numpy as jnp
from jax import lax
from jax.experimental import pallas as pl
from jax.experimental.pallas import tpu as pltpu
from jax.experimental.pallas import tpu_sc as plsc

_NC, _NS, _NL = 2, 16, 16
_NBUF = 7


@functools.partial(jax.jit, static_argnames=("b", "l", "d"))
def _shifted_embed(xt, table, b, l, d):
  nc, ns, nl = _NC, _NS, _NL
  nw = nc * ns
  bw = b // nw
  p = l - 1
  assert bw % 128 == 0 and d % nl == 0 and p % _NBUF == 0

  mesh = plsc.VectorSubcoreMesh(
      core_axis_name="c", subcore_axis_name="s", num_cores=nc, num_subcores=ns)

  buf_t = pltpu.VMEM((bw, d), jnp.float32)

  @functools.partial(
      pl.kernel,
      out_type=jax.ShapeDtypeStruct((l, b, d), jnp.float32),
      mesh=mesh,
      scratch_types=[
          pltpu.VMEM((l, bw), jnp.int32),
          [buf_t] * _NBUF,
          [pltpu.SemaphoreType.DMA] * _NBUF,
          [pltpu.SemaphoreType.DMA] * _NBUF,
      ],
      compiler_params=pltpu.CompilerParams(needs_layout_passes=False),
  )
  def k(xt_hbm, table_hbm, z_hbm, out_hbm, xv, bufs, gsems, osems):
    wid = lax.axis_index("s") * nc + lax.axis_index("c")
    b0 = wid * bw

    def _plane_dst(li):
      return out_hbm.at[li].at[pl.ds(b0, bw)]

    pltpu.async_copy(z_hbm, bufs[0], gsems[0])
    pltpu.sync_copy(xt_hbm.at[:, pl.ds(b0, bw)], xv)
    pltpu.make_async_copy(z_hbm, bufs[0], gsems[0]).wait()
    pltpu.async_copy(bufs[0], _plane_dst(l - 1), osems[0])

    def fire(li, q):
      pltpu.async_copy(table_hbm.at[xv.at[li + 1]], bufs[q], gsems[q])

    pltpu.make_async_copy(bufs[0], _plane_dst(l - 1), osems[0]).wait()
    for q in range(_NBUF):
      fire(q, q)

    def body(t, _):
      for q in range(_NBUF):
        j = t * _NBUF + q
        pltpu.make_async_copy(table_hbm.at[xv.at[0]], bufs[q],
                              gsems[q]).wait()
        pltpu.async_copy(bufs[q], _plane_dst(j), osems[q])
      for q in range(_NBUF):
        j = t * _NBUF + q
        pltpu.make_async_copy(bufs[q], _plane_dst(j), osems[q]).wait()

        @pl.when(j + _NBUF < p)
        def _():
          fire(j + _NBUF, q)

      return _

    lax.fori_loop(0, p // _NBUF, body, None)

  return k(xt, table, jnp.zeros((b // nw, d), jnp.float32))


def kernel(x, table):
  b, l = x.shape
  v, d = table.shape
  out = _shifted_embed(x.astype(jnp.int32).T, table, b, l, d)
  return out.transpose(1, 0, 2)


# speedup vs baseline: 10.7762x; 1.0107x over previous
import functools

import jax
import jax.numpy as jnp
from jax import lax
from jax.experimental import pallas as pl
from jax.experimental.pallas import tpu as pltpu
from jax.experimental.pallas import tpu_sc as plsc

_NC, _NS, _NL = 2, 16, 16
_NBUF = 6


@functools.partial(jax.jit, static_argnames=("b", "l", "d"))
def _shifted_embed(xt, table, b, l, d):
  nc, ns, nl = _NC, _NS, _NL
  nw = nc * ns
  bw = b // nw
  p = l - 1
  assert bw % 128 == 0 and d % nl == 0 and _NBUF + 2 <= 8 <= l

  mesh = plsc.VectorSubcoreMesh(
      core_axis_name="c", subcore_axis_name="s", num_cores=nc, num_subcores=ns)

  buf_t = pltpu.VMEM((bw, d), jnp.float32)

  @functools.partial(
      pl.kernel,
      out_type=jax.ShapeDtypeStruct((l, b, d), jnp.float32),
      mesh=mesh,
      scratch_types=[
          pltpu.VMEM((l, bw), jnp.int32),
          [buf_t] * (_NBUF + 1),
          [pltpu.SemaphoreType.DMA] * (_NBUF + 1),
          [pltpu.SemaphoreType.DMA] * _NBUF,
      ],
      compiler_params=pltpu.CompilerParams(needs_layout_passes=False),
  )
  def k(xt_hbm, table_hbm, z_hbm, out_hbm, xv, bufs, gsems, osems):
    wid = lax.axis_index("s") * nc + lax.axis_index("c")
    b0 = wid * bw
    zbuf, zsem = bufs[_NBUF], gsems[_NBUF]

    def _plane_dst(li):
      return out_hbm.at[li].at[pl.ds(b0, bw)]

    pltpu.async_copy(z_hbm, zbuf, zsem)
    pltpu.sync_copy(xt_hbm.at[pl.ds(0, 8)].at[:, pl.ds(b0, bw)],
                    xv.at[pl.ds(0, 8)])

    def fire(li, q):
      pltpu.async_copy(table_hbm.at[xv.at[li + 1]], bufs[q], gsems[q])

    for q in range(_NBUF):
      fire(q, q)

    pltpu.sync_copy(xt_hbm.at[pl.ds(8, l - 8)].at[:, pl.ds(b0, bw)],
                    xv.at[pl.ds(8, l - 8)])
    pltpu.make_async_copy(z_hbm, zbuf, zsem).wait()
    pltpu.async_copy(zbuf, _plane_dst(l - 1), zsem)

    def body(t, _):
      for q in range(_NBUF):
        j = t * _NBUF + q
        pltpu.make_async_copy(table_hbm.at[xv.at[0]], bufs[q],
                              gsems[q]).wait()
        pltpu.async_copy(bufs[q], _plane_dst(j), osems[q])
      for q in range(_NBUF):
        j = t * _NBUF + q
        pltpu.make_async_copy(bufs[q], _plane_dst(j), osems[q]).wait()

        @pl.when(j + _NBUF < p)
        def _():
          fire(j + _NBUF, q)

      return _

    lax.fori_loop(0, p // _NBUF, body, None)

    pltpu.make_async_copy(table_hbm.at[xv.at[0]], bufs[0], gsems[0]).wait()
    pltpu.sync_copy(bufs[0], _plane_dst(p - 1))
    pltpu.make_async_copy(zbuf, _plane_dst(l - 1), zsem).wait()

  return k(xt, table, jnp.zeros((b // nw, d), jnp.float32))


def kernel(x, table):
  b, l = x.shape
  v, d = table.shape
  out = _shifted_embed(x.astype(jnp.int32).T, table, b, l, d)
  return out.transpose(1, 0, 2)
